# baseline (speedup 1.0000x reference)
import functools

import jax
import jax.numpy as jnp
from jax import lax
from jax.experimental import pallas as pl
from jax.experimental.pallas import tpu as pltpu
from jax.experimental.pallas import tpu_sc as plsc

EPS = 1e-9
NC, NS, L = 2, 16, 16
CHUNK = 512
NHID = 64


def _dyn_splat(v, i):
    idx = jnp.full((L,), i, jnp.int32)
    return lax.gather(
        v, idx[:, None],
        lax.GatherDimensionNumbers(offset_dims=(), collapsed_slice_dims=(0,),
                                   start_index_map=(0,)),
        (1,), mode=lax.GatherScatterMode.PROMISE_IN_BOUNDS)


def _chunk_ranges(total, maxc):
    offs, off = [], 0
    while off < total:
        size = maxc
        while size > total - off:
            size //= 2
        offs.append((off, size))
        off += size
    return offs


def _qrsqrt(x):
    i = lax.bitcast_convert_type(x, jnp.int32)
    i = jnp.int32(0x5F3759DF) - lax.shift_right_logical(i, 1)
    y = lax.bitcast_convert_type(i, jnp.float32)
    for _ in range(3):
        y = y * (1.5 - 0.5 * x * y * y)
    return y


@functools.lru_cache(maxsize=None)
def _make_fused_kernel(mode, SP, MID, TP, TOT):
    nch = TOT // (NS * CHUNK)
    spr = SP // NS
    SD = TP if mode == "gcn" else MID
    secs = [SP // NS, SD // NS, TP // NS] + ([MID // NS] if MID else [])
    BN = max(secs)
    mesh = plsc.VectorSubcoreMesh(core_axis_name="c", subcore_axis_name="s")

    scratch = [
        pltpu.VMEM((CHUNK,), jnp.int32),
        pltpu.VMEM((CHUNK,), jnp.int32),
        pltpu.VMEM((CHUNK,), jnp.float32),
        pltpu.VMEM((CHUNK,), jnp.float32),
        pltpu.VMEM((CHUNK,), jnp.float32),
        pltpu.VMEM((CHUNK, NHID), jnp.float32),
        pltpu.VMEM((BN,), jnp.float32),
        pltpu.VMEM_SHARED((SP, NHID), jnp.float32),
        pltpu.VMEM_SHARED((TP if mode == "gcn" else MID, NHID),
                          jnp.float32),
        pltpu.VMEM_SHARED((SP,), jnp.float32),
        pltpu.VMEM_SHARED((SD,), jnp.float32),
        pltpu.SemaphoreType.DMA,
    ]

    @functools.partial(
        pl.kernel,
        out_type=jax.ShapeDtypeStruct((NC * TP, NHID), jnp.float32),
        mesh=mesh,
        compiler_params=pltpu.CompilerParams(
            needs_layout_passes=False, use_tc_tiling_on_sc=False),
        scratch_types=scratch,
    )
    def kern(x_hbm, g_hbm, s_hbm, w_hbm, out_hbm, *refs):
        (g_buf, s_buf, w_buf, gs_v, ss_v, rows, bounce,
         xA, B, degG, degS, sem) = refs
        cid = lax.axis_index("c")
        sid = lax.axis_index("s")

        def zero_vec(buf, total):
            def body(k, c):
                buf[pl.ds(k * L, L)] = jnp.zeros((L,), jnp.float32)
                return c
            lax.fori_loop(0, total // L, body, 0)

        def zero_rows():
            def body(k, c):
                for j in range(NHID // L):
                    rows[k, pl.ds(j * L, L)] = jnp.zeros((L,), jnp.float32)
                return c
            lax.fori_loop(0, CHUNK, body, 0)

        def zero_table(tab, rows_n):
            for off, size in _chunk_ranges(rows_n, CHUNK):
                pltpu.sync_copy(rows.at[pl.ds(0, size)],
                                tab.at[pl.ds(sid * rows_n + off, size)])

        def zero_scalar(tab, n_sec):
            pltpu.sync_copy(bounce.at[pl.ds(0, n_sec)],
                            tab.at[pl.ds(sid * n_sec, n_sec)])

        def transform(tab, n_sec, fn):
            pltpu.sync_copy(tab.at[pl.ds(sid * n_sec, n_sec)],
                            bounce.at[pl.ds(0, n_sec)])

            def body(k, c):
                v = bounce[pl.ds(k * L, L)]
                bounce[pl.ds(k * L, L)] = fn(v + EPS)
                return c
            lax.fori_loop(0, n_sec // L, body, 0)
            pltpu.sync_copy(bounce.at[pl.ds(0, n_sec)],
                            tab.at[pl.ds(sid * n_sec, n_sec)])

        def feature_pass(src_tab, dst_tab, swap, use_gs, ssT):
            def body(it, c):
                base = cid * TOT + (it * NS + sid) * CHUNK
                pltpu.sync_copy(g_hbm.at[pl.ds(base, CHUNK)], g_buf)
                pltpu.sync_copy(s_hbm.at[pl.ds(base, CHUNK)], s_buf)
                pltpu.sync_copy(w_hbm.at[pl.ds(base, CHUNK)], w_buf)
                gi = s_buf if swap else g_buf
                si = g_buf if swap else s_buf
                cp = pltpu.async_copy(src_tab.at[gi], rows, sem)
                if use_gs:
                    pltpu.sync_copy(degG.at[gi], gs_v)
                pltpu.sync_copy(ssT.at[si], ss_v)
                cp.wait()

                def scale(k, c2):
                    b16 = k * L
                    wv = w_buf[pl.ds(b16, L)] * ss_v[pl.ds(b16, L)]
                    if use_gs:
                        wv = wv * gs_v[pl.ds(b16, L)]
                    for i in range(L):
                        spl = _dyn_splat(wv, i)
                        for j in range(NHID // L):
                            sl = pl.ds(j * L, L)
                            rows[b16 + i, sl] = rows[b16 + i, sl] * spl
                    return c2

                lax.fori_loop(0, CHUNK // L, scale, 0)
                pltpu.sync_copy(rows, dst_tab.at[si], add=True)
                return c
            lax.fori_loop(0, nch, body, 0)

        def leaky_sweep(tab, rows_n):
            for off, size in _chunk_ranges(rows_n, CHUNK):
                pltpu.sync_copy(tab.at[pl.ds(sid * rows_n + off, size)],
                                rows.at[pl.ds(0, size)])

                def body(r, c):
                    for j in range(NHID // L):
                        sl = pl.ds(j * L, L)
                        v = rows[r, sl]
                        rows[r, sl] = jnp.where(v > 0, v, 0.2 * v)
                    return c
                lax.fori_loop(0, size, body, 0)
                pltpu.sync_copy(rows.at[pl.ds(0, size)],
                                tab.at[pl.ds(sid * rows_n + off, size)])

        zero_vec(bounce, BN)
        zero_scalar(degG, SP // NS)
        zero_scalar(degS, SD // NS)
        for off, size in _chunk_ranges(spr, CHUNK):
            pltpu.sync_copy(x_hbm.at[pl.ds(cid * SP + sid * spr + off, size)],
                            rows.at[pl.ds(0, size)])
            pltpu.sync_copy(rows.at[pl.ds(0, size)],
                            xA.at[pl.ds(sid * spr + off, size)])
        zero_rows()
        zero_table(B, (TP if mode == "gcn" else MID) // NS)
        plsc.subcore_barrier()

        def deg_body(it, c):
            base = cid * TOT + (it * NS + sid) * CHUNK
            pltpu.sync_copy(g_hbm.at[pl.ds(base, CHUNK)], g_buf)
            pltpu.sync_copy(s_hbm.at[pl.ds(base, CHUNK)], s_buf)
            pltpu.sync_copy(w_hbm.at[pl.ds(base, CHUNK)], w_buf)
            pltpu.sync_copy(w_buf, degG.at[g_buf], add=True)
            pltpu.sync_copy(w_buf, degS.at[s_buf], add=True)
            return c
        lax.fori_loop(0, nch, deg_body, 0)
        plsc.subcore_barrier()

        norm = _qrsqrt if mode == "gcn" else (lambda v: 1.0 / v)
        transform(degG, SP // NS, norm)
        transform(degS, SD // NS, norm)
        plsc.subcore_barrier()

        if mode == "gcn":
            feature_pass(xA, B, swap=False, use_gs=True, ssT=degS)
            out_tab = B
        elif mode == "he":
            feature_pass(xA, B, swap=False, use_gs=False, ssT=degS)
            plsc.subcore_barrier()
            zero_rows()
            zero_table(xA, SP // NS)
            plsc.subcore_barrier()
            feature_pass(B, xA, swap=True, use_gs=False, ssT=degG)
            out_tab = xA
        else:
            feature_pass(xA, B, swap=False, use_gs=False, ssT=degS)
            plsc.subcore_barrier()
            zero_rows()
            zero_table(xA, SP // NS)
            plsc.subcore_barrier()
            feature_pass(B, xA, swap=True, use_gs=False, ssT=degG)
            plsc.subcore_barrier()
            leaky_sweep(xA, SP // NS)
            zero_rows()
            zero_table(B, MID // NS)
            plsc.subcore_barrier()
            feature_pass(xA, B, swap=False, use_gs=False, ssT=degS)
            out_tab = B

        plsc.subcore_barrier()
        rps = TP // NS
        for off, size in _chunk_ranges(rps, CHUNK):
            pltpu.sync_copy(out_tab.at[pl.ds(sid * rps + off, size)],
                            rows.at[pl.ds(0, size)])
            pltpu.sync_copy(
                rows.at[pl.ds(0, size)],
                out_hbm.at[pl.ds(cid * TP + sid * rps + off, size)])

    return kern


def _pad1(g, s, bw, SP, SD, TOT):
    npad = TOT - g.shape[0]
    if npad:
        ar = jnp.arange(npad, dtype=jnp.int32)
        g = jnp.concatenate([g, ar % SP])
        s = jnp.concatenate([s, ar % SD])
        bw = jnp.concatenate([bw, jnp.zeros((npad,), bw.dtype)])
    return g, s, bw


def _xtab(x1, x2, P):
    o1 = jnp.zeros((P, x1.shape[1]), x1.dtype).at[:x1.shape[0]].set(x1)
    o2 = jnp.zeros((P, x2.shape[1]), x2.dtype).at[:x2.shape[0]].set(x2)
    return jnp.concatenate([o1, o2], axis=0)


def _fused(mode, x1, x2, pairs1, pairs2, SP, MID, TP, TOT):
    SD = TP if mode == "gcn" else MID
    g1, s1, w1 = _pad1(*pairs1, SP, SD, TOT)
    g2, s2, w2 = _pad1(*pairs2, SP, SD, TOT)
    g = jnp.concatenate([g1, g2])
    s = jnp.concatenate([s1, s2])
    w = jnp.concatenate([w1, w2])
    x = _xtab(x1, x2, SP)
    kern = _make_fused_kernel(mode, SP, MID, TP, TOT)
    return kern(x, g, s, w)


def _leaky(x):
    return jnp.where(x > 0, x, 0.2 * x)


def _readout(x, Wr):
    m = jnp.mean(x, axis=0, keepdims=True)
    gate = jax.nn.sigmoid(x @ Wr @ m.T)
    return jnp.sum(gate * x, axis=0, keepdims=True)


def _cross(x1, x2, W):
    a12 = jax.nn.softmax((x1 @ W) @ x2.T, axis=1)
    a21 = jax.nn.softmax((x2 @ W) @ x1.T, axis=1)
    return a12 @ x2, a21 @ x1


def _pool(ef, k, p):
    score = jnp.tanh(ef @ p / (jnp.linalg.norm(p) + EPS))
    vals, idx = lax.top_k(score, k)
    pooled = ef[idx] * vals[:, None]
    num = ef.shape[0]
    mapping = jnp.zeros((num,), jnp.int32).at[idx].set(
        jnp.arange(k, dtype=jnp.int32))
    keep = jnp.zeros((num,), ef.dtype).at[idx].set(1.0)
    return pooled, mapping, keep


def kernel(features_1, edge_index_1, edge_attr_1, batch_1, features_2,
           edge_index_2, edge_attr_2, batch_2, W0, b0, W1, W2, W3, Wc1, Wc2,
           Wc3, p1, p2, p3, Wr0, Wr1, Wr2, Wr3, Wm1, bm1, Wm2, bm2):
    n = features_1.shape[0]
    K1 = int(0.2 * n); K2 = K1 // 2; K3 = K2 // 2
    NP = -(-n // 1024) * 1024
    KP1 = -(-K1 // 1024) * 1024
    KP2 = -(-K2 // 1024) * 1024
    E_ = edge_index_1.shape[1]
    M_ = E_ + n
    GTOT = -(-E_ // (NS * CHUNK)) * (NS * CHUNK)
    FTOT = -(-M_ // (NS * CHUNK)) * (NS * CHUNK)
    src1, dst1 = edge_index_1[0], edge_index_1[1]
    src2, dst2 = edge_index_2[0], edge_index_2[1]
    ew1, ew2 = edge_attr_1, edge_attr_2

    h1 = features_1 @ W0
    h2 = features_2 @ W0
    out = _fused("gcn", h1, h2, (src1, dst1, ew1), (src2, dst2, ew2),
                 NP, None, NP, GTOT)
    f1 = _leaky(out[:n] + b0)
    f2 = _leaky(out[NP:NP + n] + b0)
    s0 = jnp.concatenate([_readout(f1, Wr0), _readout(f2, Wr0)], axis=1)

    ar_n = jnp.arange(n, dtype=jnp.int32)
    n1 = jnp.concatenate([src1, ar_n]); h1i = jnp.concatenate([dst1, ar_n])
    a1 = jnp.concatenate([ew1, jnp.ones((n,), jnp.float32)])
    n2 = jnp.concatenate([src2, ar_n]); h2i = jnp.concatenate([dst2, ar_n])
    a2 = jnp.concatenate([ew2, jnp.ones((n,), jnp.float32)])

    hh1 = f1 @ W1; hh2 = f2 @ W1
    ef = _fused("hg", hh1, hh2, (n1, h1i, a1), (n2, h2i, a2),
                NP, NP, NP, FTOT)
    ef1 = ef[:n]; ef2 = ef[NP:NP + n]

    e1, map1, keep1 = _pool(ef1, K1, p1)
    e2, map2, keep2 = _pool(ef2, K1, p1)
    h1p = map1[h1i]; a1p = a1 * keep1[h1i]
    h2p = map2[h2i]; a2p = a2 * keep2[h2i]
    x1, x2 = _cross(e1, e2, Wc1)
    s1 = jnp.concatenate([_readout(x1, Wr1), _readout(x2, Wr1)], axis=1)

    def he_layer(x1, x2, h1p, a1p, h2p, a2p, K, KP, W):
        out = _fused("he", x1, x2, (h1p, n1, a1p), (h2p, n2, a2p),
                     KP, NP, KP, FTOT)
        o1 = _leaky(out[:K] @ W)
        o2 = _leaky(out[KP:KP + K] @ W)
        return o1, o2

    g1o, g2o = he_layer(x1, x2, h1p, a1p, h2p, a2p, K1, KP1, W2)
    e1, m1b, k1b = _pool(g1o, K2, p2)
    e2, m2b, k2b = _pool(g2o, K2, p2)
    h1p2 = m1b[h1p]; a1p2 = a1p * k1b[h1p]
    h2p2 = m2b[h2p]; a2p2 = a2p * k2b[h2p]
    x1, x2 = _cross(e1, e2, Wc2)
    s2 = jnp.concatenate([_readout(x1, Wr2), _readout(x2, Wr2)], axis=1)

    g1o, g2o = he_layer(x1, x2, h1p2, a1p2, h2p2, a2p2, K2, KP2, W3)
    e1, _, _ = _pool(g1o, K3, p3)
    e2, _, _ = _pool(g2o, K3, p3)
    x1, x2 = _cross(e1, e2, Wc3)
    s3 = jnp.concatenate([_readout(x1, Wr3), _readout(x2, Wr3)], axis=1)

    scores = jnp.concatenate([s0, s1, s2, s3], axis=1)
    hmid = _leaky(scores @ Wm1 + bm1)
    return hmid @ Wm2 + bm2


# speedup vs baseline: 1.0595x; 1.0595x over previous
import functools

import jax
import jax.numpy as jnp
from jax import lax
from jax.experimental import pallas as pl
from jax.experimental.pallas import tpu as pltpu
from jax.experimental.pallas import tpu_sc as plsc

EPS = 1e-9
NC, NS, L = 2, 16, 16
CHUNK = 512
NHID = 64


def _dyn_splat(v, i):
    idx = jnp.full((L,), i, jnp.int32)
    return lax.gather(
        v, idx[:, None],
        lax.GatherDimensionNumbers(offset_dims=(), collapsed_slice_dims=(0,),
                                   start_index_map=(0,)),
        (1,), mode=lax.GatherScatterMode.PROMISE_IN_BOUNDS)


def _chunk_ranges(total, maxc):
    offs, off = [], 0
    while off < total:
        size = maxc
        while size > total - off:
            size //= 2
        offs.append((off, size))
        off += size
    return offs


def _qrsqrt(x):
    i = lax.bitcast_convert_type(x, jnp.int32)
    i = jnp.int32(0x5F3759DF) - lax.shift_right_logical(i, 1)
    y = lax.bitcast_convert_type(i, jnp.float32)
    for _ in range(3):
        y = y * (1.5 - 0.5 * x * y * y)
    return y


@functools.lru_cache(maxsize=None)
def _make_fused_kernel(mode, SP, MID, TP, TOT):
    nch = TOT // (NS * CHUNK)
    spr = SP // NS
    SD = TP if mode == "gcn" else MID
    secs = [SP // NS, SD // NS, TP // NS] + ([MID // NS] if MID else [])
    BN = max(secs)
    mesh = plsc.VectorSubcoreMesh(core_axis_name="c", subcore_axis_name="s")

    scratch = [
        pltpu.VMEM((CHUNK,), jnp.int32),
        pltpu.VMEM((CHUNK,), jnp.int32),
        pltpu.VMEM((CHUNK,), jnp.float32),
        pltpu.VMEM((CHUNK,), jnp.float32),
        pltpu.VMEM((CHUNK,), jnp.float32),
        pltpu.VMEM((CHUNK, NHID), jnp.float32),
        pltpu.VMEM((BN,), jnp.float32),
        pltpu.VMEM_SHARED((SP, NHID), jnp.float32),
        pltpu.VMEM_SHARED((TP if mode == "gcn" else MID, NHID),
                          jnp.float32),
        pltpu.VMEM_SHARED((SP,), jnp.float32),
        pltpu.VMEM_SHARED((SD,), jnp.float32),
        pltpu.SemaphoreType.DMA,
    ]

    @functools.partial(
        pl.kernel,
        out_type=jax.ShapeDtypeStruct((NC * TP, NHID), jnp.float32),
        mesh=mesh,
        compiler_params=pltpu.CompilerParams(
            needs_layout_passes=False, use_tc_tiling_on_sc=False),
        scratch_types=scratch,
    )
    def kern(x_hbm, g_hbm, s_hbm, w_hbm, out_hbm, *refs):
        (g_buf, s_buf, w_buf, gs_v, ss_v, rows, bounce,
         xA, B, degG, degS, sem) = refs
        cid = lax.axis_index("c")
        sid = lax.axis_index("s")

        def zero_vec(buf, total):
            def body(k, c):
                buf[pl.ds(k * L, L)] = jnp.zeros((L,), jnp.float32)
                return c
            lax.fori_loop(0, total // L, body, 0)

        def zero_rows():
            def body(k, c):
                for j in range(NHID // L):
                    rows[k, pl.ds(j * L, L)] = jnp.zeros((L,), jnp.float32)
                return c
            lax.fori_loop(0, CHUNK, body, 0)

        def zero_table(tab, rows_n):
            for off, size in _chunk_ranges(rows_n, CHUNK):
                pltpu.sync_copy(rows.at[pl.ds(0, size)],
                                tab.at[pl.ds(sid * rows_n + off, size)])

        def zero_scalar(tab, n_sec):
            pltpu.sync_copy(bounce.at[pl.ds(0, n_sec)],
                            tab.at[pl.ds(sid * n_sec, n_sec)])

        def transform(tab, n_sec, fn):
            pltpu.sync_copy(tab.at[pl.ds(sid * n_sec, n_sec)],
                            bounce.at[pl.ds(0, n_sec)])

            def body(k, c):
                v = bounce[pl.ds(k * L, L)]
                bounce[pl.ds(k * L, L)] = fn(v + EPS)
                return c
            lax.fori_loop(0, n_sec // L, body, 0)
            pltpu.sync_copy(bounce.at[pl.ds(0, n_sec)],
                            tab.at[pl.ds(sid * n_sec, n_sec)])

        def feature_pass(src_tab, dst_tab, swap, use_gs, ssT):
            def body(it, c):
                base = cid * TOT + (it * NS + sid) * CHUNK
                pltpu.sync_copy(g_hbm.at[pl.ds(base, CHUNK)], g_buf)
                pltpu.sync_copy(s_hbm.at[pl.ds(base, CHUNK)], s_buf)
                pltpu.sync_copy(w_hbm.at[pl.ds(base, CHUNK)], w_buf)
                gi = s_buf if swap else g_buf
                si = g_buf if swap else s_buf
                cp = pltpu.async_copy(src_tab.at[gi], rows, sem)
                if use_gs:
                    pltpu.sync_copy(degG.at[gi], gs_v)
                pltpu.sync_copy(ssT.at[si], ss_v)
                cp.wait()

                def scale(k, c2):
                    b16 = k * L
                    wv = w_buf[pl.ds(b16, L)] * ss_v[pl.ds(b16, L)]
                    if use_gs:
                        wv = wv * gs_v[pl.ds(b16, L)]
                    for i in range(L):
                        spl = _dyn_splat(wv, i)
                        for j in range(NHID // L):
                            sl = pl.ds(j * L, L)
                            rows[b16 + i, sl] = rows[b16 + i, sl] * spl
                    return c2

                lax.fori_loop(0, CHUNK // L, scale, 0)
                pltpu.sync_copy(rows, dst_tab.at[si], add=True)
                return c
            lax.fori_loop(0, nch, body, 0)

        def leaky_sweep(tab, rows_n):
            for off, size in _chunk_ranges(rows_n, CHUNK):
                pltpu.sync_copy(tab.at[pl.ds(sid * rows_n + off, size)],
                                rows.at[pl.ds(0, size)])

                def body(r, c):
                    for j in range(NHID // L):
                        sl = pl.ds(j * L, L)
                        v = rows[r, sl]
                        rows[r, sl] = jnp.where(v > 0, v, 0.2 * v)
                    return c
                lax.fori_loop(0, size, body, 0)
                pltpu.sync_copy(rows.at[pl.ds(0, size)],
                                tab.at[pl.ds(sid * rows_n + off, size)])

        zero_vec(bounce, BN)
        zero_scalar(degG, SP // NS)
        zero_scalar(degS, SD // NS)
        for off, size in _chunk_ranges(spr, CHUNK):
            pltpu.sync_copy(x_hbm.at[pl.ds(cid * SP + sid * spr + off, size)],
                            rows.at[pl.ds(0, size)])
            pltpu.sync_copy(rows.at[pl.ds(0, size)],
                            xA.at[pl.ds(sid * spr + off, size)])
        zero_rows()
        zero_table(B, (TP if mode == "gcn" else MID) // NS)
        plsc.subcore_barrier()

        def deg_body(it, c):
            base = cid * TOT + (it * NS + sid) * CHUNK
            pltpu.sync_copy(g_hbm.at[pl.ds(base, CHUNK)], g_buf)
            pltpu.sync_copy(s_hbm.at[pl.ds(base, CHUNK)], s_buf)
            pltpu.sync_copy(w_hbm.at[pl.ds(base, CHUNK)], w_buf)
            pltpu.sync_copy(w_buf, degG.at[g_buf], add=True)
            pltpu.sync_copy(w_buf, degS.at[s_buf], add=True)
            return c
        lax.fori_loop(0, nch, deg_body, 0)
        plsc.subcore_barrier()

        norm = _qrsqrt if mode == "gcn" else (lambda v: 1.0 / v)
        transform(degG, SP // NS, norm)
        transform(degS, SD // NS, norm)
        plsc.subcore_barrier()

        if mode == "gcn":
            feature_pass(xA, B, swap=False, use_gs=True, ssT=degS)
            out_tab = B
        elif mode == "he":
            feature_pass(xA, B, swap=False, use_gs=False, ssT=degS)
            plsc.subcore_barrier()
            zero_rows()
            zero_table(xA, SP // NS)
            plsc.subcore_barrier()
            feature_pass(B, xA, swap=True, use_gs=False, ssT=degG)
            out_tab = xA
        else:
            feature_pass(xA, B, swap=False, use_gs=False, ssT=degS)
            plsc.subcore_barrier()
            zero_rows()
            zero_table(xA, SP // NS)
            plsc.subcore_barrier()
            feature_pass(B, xA, swap=True, use_gs=False, ssT=degG)
            plsc.subcore_barrier()
            leaky_sweep(xA, SP // NS)
            zero_rows()
            zero_table(B, MID // NS)
            plsc.subcore_barrier()
            feature_pass(xA, B, swap=False, use_gs=False, ssT=degS)
            out_tab = B

        plsc.subcore_barrier()
        rps = TP // NS
        for off, size in _chunk_ranges(rps, CHUNK):
            pltpu.sync_copy(out_tab.at[pl.ds(sid * rps + off, size)],
                            rows.at[pl.ds(0, size)])
            pltpu.sync_copy(
                rows.at[pl.ds(0, size)],
                out_hbm.at[pl.ds(cid * TP + sid * rps + off, size)])

    return kern


def _pad1(g, s, bw, SP, SD, TOT):
    npad = TOT - g.shape[0]
    if npad:
        ar = jnp.arange(npad, dtype=jnp.int32)
        g = jnp.concatenate([g, ar % SP])
        s = jnp.concatenate([s, ar % SD])
        bw = jnp.concatenate([bw, jnp.zeros((npad,), bw.dtype)])
    return g, s, bw


def _xtab(x1, x2, P):
    o1 = jnp.zeros((P, x1.shape[1]), x1.dtype).at[:x1.shape[0]].set(x1)
    o2 = jnp.zeros((P, x2.shape[1]), x2.dtype).at[:x2.shape[0]].set(x2)
    return jnp.concatenate([o1, o2], axis=0)


def _fused(mode, x1, x2, pairs1, pairs2, SP, MID, TP, TOT):
    SD = TP if mode == "gcn" else MID
    g1, s1, w1 = _pad1(*pairs1, SP, SD, TOT)
    g2, s2, w2 = _pad1(*pairs2, SP, SD, TOT)
    g = jnp.concatenate([g1, g2])
    s = jnp.concatenate([s1, s2])
    w = jnp.concatenate([w1, w2])
    x = _xtab(x1, x2, SP)
    kern = _make_fused_kernel(mode, SP, MID, TP, TOT)
    return kern(x, g, s, w)


def _leaky(x):
    return jnp.where(x > 0, x, 0.2 * x)


def _readout(x, Wr):
    m = jnp.mean(x, axis=0, keepdims=True)
    gate = jax.nn.sigmoid(x @ Wr @ m.T)
    return jnp.sum(gate * x, axis=0, keepdims=True)


def _cross(x1, x2, W):
    a12 = jax.nn.softmax((x1 @ W) @ x2.T, axis=1)
    a21 = jax.nn.softmax((x2 @ W) @ x1.T, axis=1)
    return a12 @ x2, a21 @ x1


def _pool(ef, k, p, kp):
    score = jnp.tanh(ef @ p / (jnp.linalg.norm(p) + EPS))
    vals, idx = lax.top_k(score, k)
    pooled = ef[idx] * vals[:, None]
    num = ef.shape[0]
    mapping = (jnp.arange(num, dtype=jnp.int32) % kp).at[idx].set(
        jnp.arange(k, dtype=jnp.int32))
    keep = jnp.zeros((num,), ef.dtype).at[idx].set(1.0)
    return pooled, mapping, keep


def kernel(features_1, edge_index_1, edge_attr_1, batch_1, features_2,
           edge_index_2, edge_attr_2, batch_2, W0, b0, W1, W2, W3, Wc1, Wc2,
           Wc3, p1, p2, p3, Wr0, Wr1, Wr2, Wr3, Wm1, bm1, Wm2, bm2):
    n = features_1.shape[0]
    K1 = int(0.2 * n); K2 = K1 // 2; K3 = K2 // 2
    NP = -(-n // 1024) * 1024
    KP1 = -(-K1 // 1024) * 1024
    KP2 = -(-K2 // 1024) * 1024
    E_ = edge_index_1.shape[1]
    M_ = E_ + n
    GTOT = -(-E_ // (NS * CHUNK)) * (NS * CHUNK)
    FTOT = -(-M_ // (NS * CHUNK)) * (NS * CHUNK)
    src1, dst1 = edge_index_1[0], edge_index_1[1]
    src2, dst2 = edge_index_2[0], edge_index_2[1]
    ew1, ew2 = edge_attr_1, edge_attr_2

    h1 = features_1 @ W0
    h2 = features_2 @ W0
    out = _fused("gcn", h1, h2, (src1, dst1, ew1), (src2, dst2, ew2),
                 NP, None, NP, GTOT)
    f1 = _leaky(out[:n] + b0)
    f2 = _leaky(out[NP:NP + n] + b0)
    s0 = jnp.concatenate([_readout(f1, Wr0), _readout(f2, Wr0)], axis=1)

    ar_n = jnp.arange(n, dtype=jnp.int32)
    n1 = jnp.concatenate([src1, ar_n]); h1i = jnp.concatenate([dst1, ar_n])
    a1 = jnp.concatenate([ew1, jnp.ones((n,), jnp.float32)])
    n2 = jnp.concatenate([src2, ar_n]); h2i = jnp.concatenate([dst2, ar_n])
    a2 = jnp.concatenate([ew2, jnp.ones((n,), jnp.float32)])

    hh1 = f1 @ W1; hh2 = f2 @ W1
    ef = _fused("hg", hh1, hh2, (n1, h1i, a1), (n2, h2i, a2),
                NP, NP, NP, FTOT)
    ef1 = ef[:n]; ef2 = ef[NP:NP + n]

    e1, map1, keep1 = _pool(ef1, K1, p1, KP1)
    e2, map2, keep2 = _pool(ef2, K1, p1, KP1)
    h1p = map1[h1i]; a1p = a1 * keep1[h1i]
    h2p = map2[h2i]; a2p = a2 * keep2[h2i]
    x1, x2 = _cross(e1, e2, Wc1)
    s1 = jnp.concatenate([_readout(x1, Wr1), _readout(x2, Wr1)], axis=1)

    def he_layer(x1, x2, h1p, a1p, h2p, a2p, K, KP, W):
        out = _fused("he", x1, x2, (h1p, n1, a1p), (h2p, n2, a2p),
                     KP, NP, KP, FTOT)
        o1 = _leaky(out[:K] @ W)
        o2 = _leaky(out[KP:KP + K] @ W)
        return o1, o2

    g1o, g2o = he_layer(x1, x2, h1p, a1p, h2p, a2p, K1, KP1, W2)
    e1, m1b, k1b = _pool(g1o, K2, p2, KP2)
    e2, m2b, k2b = _pool(g2o, K2, p2, KP2)

    def _ext(m, keep, total, kp):
        num = m.shape[0]
        extra = jnp.arange(num, total, dtype=jnp.int32) % kp
        return (jnp.concatenate([m, extra]),
                jnp.concatenate([keep, jnp.zeros((total - num,), keep.dtype)]))

    m1b, k1b = _ext(m1b, k1b, KP1, KP2)
    m2b, k2b = _ext(m2b, k2b, KP1, KP2)
    h1p2 = m1b[h1p]; a1p2 = a1p * k1b[h1p]
    h2p2 = m2b[h2p]; a2p2 = a2p * k2b[h2p]
    x1, x2 = _cross(e1, e2, Wc2)
    s2 = jnp.concatenate([_readout(x1, Wr2), _readout(x2, Wr2)], axis=1)

    g1o, g2o = he_layer(x1, x2, h1p2, a1p2, h2p2, a2p2, K2, KP2, W3)
    e1, _, _ = _pool(g1o, K3, p3, K3)
    e2, _, _ = _pool(g2o, K3, p3, K3)
    x1, x2 = _cross(e1, e2, Wc3)
    s3 = jnp.concatenate([_readout(x1, Wr3), _readout(x2, Wr3)], axis=1)

    scores = jnp.concatenate([s0, s1, s2, s3], axis=1)
    hmid = _leaky(scores @ Wm1 + bm1)
    return hmid @ Wm2 + bm2


# speedup vs baseline: 1.0804x; 1.0197x over previous
import functools

import jax
import jax.numpy as jnp
from jax import lax
from jax.experimental import pallas as pl
from jax.experimental.pallas import tpu as pltpu
from jax.experimental.pallas import tpu_sc as plsc

EPS = 1e-9
NC, NS, L = 2, 16, 16
CHUNK = 512
NHID = 64


def _dyn_splat(v, i):
    idx = jnp.full((L,), i, jnp.int32)
    return lax.gather(
        v, idx[:, None],
        lax.GatherDimensionNumbers(offset_dims=(), collapsed_slice_dims=(0,),
                                   start_index_map=(0,)),
        (1,), mode=lax.GatherScatterMode.PROMISE_IN_BOUNDS)


def _chunk_ranges(total, maxc):
    offs, off = [], 0
    while off < total:
        size = maxc
        while size > total - off:
            size //= 2
        offs.append((off, size))
        off += size
    return offs


def _qrsqrt(x):
    i = lax.bitcast_convert_type(x, jnp.int32)
    i = jnp.int32(0x5F3759DF) - lax.shift_right_logical(i, 1)
    y = lax.bitcast_convert_type(i, jnp.float32)
    for _ in range(3):
        y = y * (1.5 - 0.5 * x * y * y)
    return y


@functools.lru_cache(maxsize=None)
def _make_fused_kernel(mode, SP, MID, TP, TOT):
    nch = TOT // (NS * CHUNK)
    spr = SP // NS
    SD = TP if mode == "gcn" else MID
    secs = [SP // NS, SD // NS, TP // NS] + ([MID // NS] if MID else [])
    BN = max(secs)
    mesh = plsc.VectorSubcoreMesh(core_axis_name="c", subcore_axis_name="s")

    scratch = [
        pltpu.VMEM((CHUNK,), jnp.int32),
        pltpu.VMEM((CHUNK,), jnp.int32),
        pltpu.VMEM((CHUNK,), jnp.float32),
        pltpu.VMEM((CHUNK,), jnp.float32),
        pltpu.VMEM((CHUNK,), jnp.float32),
        pltpu.VMEM((CHUNK, NHID), jnp.float32),
        pltpu.VMEM((BN,), jnp.float32),
        pltpu.VMEM_SHARED((SP, NHID), jnp.float32),
        pltpu.VMEM_SHARED((TP if mode == "gcn" else MID, NHID),
                          jnp.float32),
        pltpu.VMEM_SHARED((SP,), jnp.float32),
        pltpu.VMEM_SHARED((SD,), jnp.float32),
        pltpu.SemaphoreType.DMA,
        pltpu.SemaphoreType.DMA,
        pltpu.SemaphoreType.DMA,
        pltpu.SemaphoreType.DMA,
    ]

    @functools.partial(
        pl.kernel,
        out_type=jax.ShapeDtypeStruct((NC * TP, NHID), jnp.float32),
        mesh=mesh,
        compiler_params=pltpu.CompilerParams(
            needs_layout_passes=False, use_tc_tiling_on_sc=False),
        scratch_types=scratch,
    )
    def kern(x_hbm, g_hbm, s_hbm, w_hbm, out_hbm, *refs):
        (g_buf, s_buf, w_buf, gs_v, ss_v, rows, bounce,
         xA, B, degG, degS, sem, semg, sems, semw) = refs
        cid = lax.axis_index("c")
        sid = lax.axis_index("s")

        def zero_vec(buf, total):
            def body(k, c):
                buf[pl.ds(k * L, L)] = jnp.zeros((L,), jnp.float32)
                return c
            lax.fori_loop(0, total // L, body, 0)

        def zero_rows():
            def body(k, c):
                for j in range(NHID // L):
                    rows[k, pl.ds(j * L, L)] = jnp.zeros((L,), jnp.float32)
                return c
            lax.fori_loop(0, CHUNK, body, 0)

        def zero_table(tab, rows_n):
            for off, size in _chunk_ranges(rows_n, CHUNK):
                pltpu.sync_copy(rows.at[pl.ds(0, size)],
                                tab.at[pl.ds(sid * rows_n + off, size)])

        def zero_scalar(tab, n_sec):
            pltpu.sync_copy(bounce.at[pl.ds(0, n_sec)],
                            tab.at[pl.ds(sid * n_sec, n_sec)])

        def transform(tab, n_sec, fn):
            pltpu.sync_copy(tab.at[pl.ds(sid * n_sec, n_sec)],
                            bounce.at[pl.ds(0, n_sec)])

            def body(k, c):
                v = bounce[pl.ds(k * L, L)]
                bounce[pl.ds(k * L, L)] = fn(v + EPS)
                return c
            lax.fori_loop(0, n_sec // L, body, 0)
            pltpu.sync_copy(bounce.at[pl.ds(0, n_sec)],
                            tab.at[pl.ds(sid * n_sec, n_sec)])

        def feature_pass(src_tab, dst_tab, swap, use_gs, ssT):
            def body(it, c):
                base = cid * TOT + (it * NS + sid) * CHUNK
                cpg = pltpu.async_copy(g_hbm.at[pl.ds(base, CHUNK)], g_buf,
                                       semg)
                cps = pltpu.async_copy(s_hbm.at[pl.ds(base, CHUNK)], s_buf,
                                       sems)
                cpw = pltpu.async_copy(w_hbm.at[pl.ds(base, CHUNK)], w_buf,
                                       semw)
                gi = s_buf if swap else g_buf
                si = g_buf if swap else s_buf
                cpg.wait()
                cps.wait()
                cp = pltpu.async_copy(src_tab.at[gi], rows, sem)
                if use_gs:
                    pltpu.sync_copy(degG.at[gi], gs_v)
                pltpu.sync_copy(ssT.at[si], ss_v)
                cpw.wait()
                cp.wait()

                def scale(k, c2):
                    b16 = k * L
                    wv = w_buf[pl.ds(b16, L)] * ss_v[pl.ds(b16, L)]
                    if use_gs:
                        wv = wv * gs_v[pl.ds(b16, L)]
                    for i in range(L):
                        spl = _dyn_splat(wv, i)
                        for j in range(NHID // L):
                            sl = pl.ds(j * L, L)
                            rows[b16 + i, sl] = rows[b16 + i, sl] * spl
                    return c2

                lax.fori_loop(0, CHUNK // L, scale, 0)
                pltpu.sync_copy(rows, dst_tab.at[si], add=True)
                return c
            lax.fori_loop(0, nch, body, 0)

        def leaky_sweep(tab, rows_n):
            for off, size in _chunk_ranges(rows_n, CHUNK):
                pltpu.sync_copy(tab.at[pl.ds(sid * rows_n + off, size)],
                                rows.at[pl.ds(0, size)])

                def body(r, c):
                    for j in range(NHID // L):
                        sl = pl.ds(j * L, L)
                        v = rows[r, sl]
                        rows[r, sl] = jnp.where(v > 0, v, 0.2 * v)
                    return c
                lax.fori_loop(0, size, body, 0)
                pltpu.sync_copy(rows.at[pl.ds(0, size)],
                                tab.at[pl.ds(sid * rows_n + off, size)])

        zero_vec(bounce, BN)
        zero_scalar(degG, SP // NS)
        zero_scalar(degS, SD // NS)
        for off, size in _chunk_ranges(spr, CHUNK):
            pltpu.sync_copy(x_hbm.at[pl.ds(cid * SP + sid * spr + off, size)],
                            rows.at[pl.ds(0, size)])
            pltpu.sync_copy(rows.at[pl.ds(0, size)],
                            xA.at[pl.ds(sid * spr + off, size)])
        zero_rows()
        zero_table(B, (TP if mode == "gcn" else MID) // NS)
        plsc.subcore_barrier()

        def deg_body(it, c):
            base = cid * TOT + (it * NS + sid) * CHUNK
            cpg = pltpu.async_copy(g_hbm.at[pl.ds(base, CHUNK)], g_buf, semg)
            cps = pltpu.async_copy(s_hbm.at[pl.ds(base, CHUNK)], s_buf, sems)
            cpw = pltpu.async_copy(w_hbm.at[pl.ds(base, CHUNK)], w_buf, semw)
            cpg.wait()
            cps.wait()
            cpw.wait()
            pltpu.sync_copy(w_buf, degG.at[g_buf], add=True)
            pltpu.sync_copy(w_buf, degS.at[s_buf], add=True)
            return c
        lax.fori_loop(0, nch, deg_body, 0)
        plsc.subcore_barrier()

        norm = _qrsqrt if mode == "gcn" else (lambda v: 1.0 / v)
        transform(degG, SP // NS, norm)
        transform(degS, SD // NS, norm)
        plsc.subcore_barrier()

        if mode == "gcn":
            feature_pass(xA, B, swap=False, use_gs=True, ssT=degS)
            out_tab = B
        elif mode == "he":
            feature_pass(xA, B, swap=False, use_gs=False, ssT=degS)
            plsc.subcore_barrier()
            zero_rows()
            zero_table(xA, SP // NS)
            plsc.subcore_barrier()
            feature_pass(B, xA, swap=True, use_gs=False, ssT=degG)
            out_tab = xA
        else:
            feature_pass(xA, B, swap=False, use_gs=False, ssT=degS)
            plsc.subcore_barrier()
            zero_rows()
            zero_table(xA, SP // NS)
            plsc.subcore_barrier()
            feature_pass(B, xA, swap=True, use_gs=False, ssT=degG)
            plsc.subcore_barrier()
            leaky_sweep(xA, SP // NS)
            zero_rows()
            zero_table(B, MID // NS)
            plsc.subcore_barrier()
            feature_pass(xA, B, swap=False, use_gs=False, ssT=degS)
            out_tab = B

        plsc.subcore_barrier()
        rps = TP // NS
        for off, size in _chunk_ranges(rps, CHUNK):
            pltpu.sync_copy(out_tab.at[pl.ds(sid * rps + off, size)],
                            rows.at[pl.ds(0, size)])
            pltpu.sync_copy(
                rows.at[pl.ds(0, size)],
                out_hbm.at[pl.ds(cid * TP + sid * rps + off, size)])

    return kern


def _pad1(g, s, bw, SP, SD, TOT):
    npad = TOT - g.shape[0]
    if npad:
        ar = jnp.arange(npad, dtype=jnp.int32)
        g = jnp.concatenate([g, ar % SP])
        s = jnp.concatenate([s, ar % SD])
        bw = jnp.concatenate([bw, jnp.zeros((npad,), bw.dtype)])
    return g, s, bw


def _xtab(x1, x2, P):
    o1 = jnp.zeros((P, x1.shape[1]), x1.dtype).at[:x1.shape[0]].set(x1)
    o2 = jnp.zeros((P, x2.shape[1]), x2.dtype).at[:x2.shape[0]].set(x2)
    return jnp.concatenate([o1, o2], axis=0)


def _fused(mode, x1, x2, pairs1, pairs2, SP, MID, TP, TOT):
    SD = TP if mode == "gcn" else MID
    g1, s1, w1 = _pad1(*pairs1, SP, SD, TOT)
    g2, s2, w2 = _pad1(*pairs2, SP, SD, TOT)
    g = jnp.concatenate([g1, g2])
    s = jnp.concatenate([s1, s2])
    w = jnp.concatenate([w1, w2])
    x = _xtab(x1, x2, SP)
    kern = _make_fused_kernel(mode, SP, MID, TP, TOT)
    return kern(x, g, s, w)


def _leaky(x):
    return jnp.where(x > 0, x, 0.2 * x)


def _readout(x, Wr):
    m = jnp.mean(x, axis=0, keepdims=True)
    gate = jax.nn.sigmoid(x @ Wr @ m.T)
    return jnp.sum(gate * x, axis=0, keepdims=True)


def _cross(x1, x2, W):
    a12 = jax.nn.softmax((x1 @ W) @ x2.T, axis=1)
    a21 = jax.nn.softmax((x2 @ W) @ x1.T, axis=1)
    return a12 @ x2, a21 @ x1


def _pool(ef, k, p, kp):
    score = jnp.tanh(ef @ p / (jnp.linalg.norm(p) + EPS))
    vals, idx = lax.top_k(score, k)
    pooled = ef[idx] * vals[:, None]
    num = ef.shape[0]
    mapping = (jnp.arange(num, dtype=jnp.int32) % kp).at[idx].set(
        jnp.arange(k, dtype=jnp.int32))
    keep = jnp.zeros((num,), ef.dtype).at[idx].set(1.0)
    return pooled, mapping, keep


def kernel(features_1, edge_index_1, edge_attr_1, batch_1, features_2,
           edge_index_2, edge_attr_2, batch_2, W0, b0, W1, W2, W3, Wc1, Wc2,
           Wc3, p1, p2, p3, Wr0, Wr1, Wr2, Wr3, Wm1, bm1, Wm2, bm2):
    n = features_1.shape[0]
    K1 = int(0.2 * n); K2 = K1 // 2; K3 = K2 // 2
    NP = -(-n // 1024) * 1024
    KP1 = -(-K1 // 1024) * 1024
    KP2 = -(-K2 // 1024) * 1024
    E_ = edge_index_1.shape[1]
    M_ = E_ + n
    GTOT = -(-E_ // (NS * CHUNK)) * (NS * CHUNK)
    FTOT = -(-M_ // (NS * CHUNK)) * (NS * CHUNK)
    src1, dst1 = edge_index_1[0], edge_index_1[1]
    src2, dst2 = edge_index_2[0], edge_index_2[1]
    ew1, ew2 = edge_attr_1, edge_attr_2

    h1 = features_1 @ W0
    h2 = features_2 @ W0
    out = _fused("gcn", h1, h2, (src1, dst1, ew1), (src2, dst2, ew2),
                 NP, None, NP, GTOT)
    f1 = _leaky(out[:n] + b0)
    f2 = _leaky(out[NP:NP + n] + b0)
    s0 = jnp.concatenate([_readout(f1, Wr0), _readout(f2, Wr0)], axis=1)

    ar_n = jnp.arange(n, dtype=jnp.int32)
    n1 = jnp.concatenate([src1, ar_n]); h1i = jnp.concatenate([dst1, ar_n])
    a1 = jnp.concatenate([ew1, jnp.ones((n,), jnp.float32)])
    n2 = jnp.concatenate([src2, ar_n]); h2i = jnp.concatenate([dst2, ar_n])
    a2 = jnp.concatenate([ew2, jnp.ones((n,), jnp.float32)])

    hh1 = f1 @ W1; hh2 = f2 @ W1
    ef = _fused("hg", hh1, hh2, (n1, h1i, a1), (n2, h2i, a2),
                NP, NP, NP, FTOT)
    ef1 = ef[:n]; ef2 = ef[NP:NP + n]

    e1, map1, keep1 = _pool(ef1, K1, p1, KP1)
    e2, map2, keep2 = _pool(ef2, K1, p1, KP1)
    h1p = map1[h1i]; a1p = a1 * keep1[h1i]
    h2p = map2[h2i]; a2p = a2 * keep2[h2i]
    x1, x2 = _cross(e1, e2, Wc1)
    s1 = jnp.concatenate([_readout(x1, Wr1), _readout(x2, Wr1)], axis=1)

    def he_layer(x1, x2, h1p, a1p, h2p, a2p, K, KP, W):
        out = _fused("he", x1, x2, (h1p, n1, a1p), (h2p, n2, a2p),
                     KP, NP, KP, FTOT)
        o1 = _leaky(out[:K] @ W)
        o2 = _leaky(out[KP:KP + K] @ W)
        return o1, o2

    g1o, g2o = he_layer(x1, x2, h1p, a1p, h2p, a2p, K1, KP1, W2)
    e1, m1b, k1b = _pool(g1o, K2, p2, KP2)
    e2, m2b, k2b = _pool(g2o, K2, p2, KP2)

    def _ext(m, keep, total, kp):
        num = m.shape[0]
        extra = jnp.arange(num, total, dtype=jnp.int32) % kp
        return (jnp.concatenate([m, extra]),
                jnp.concatenate([keep, jnp.zeros((total - num,), keep.dtype)]))

    m1b, k1b = _ext(m1b, k1b, KP1, KP2)
    m2b, k2b = _ext(m2b, k2b, KP1, KP2)
    h1p2 = m1b[h1p]; a1p2 = a1p * k1b[h1p]
    h2p2 = m2b[h2p]; a2p2 = a2p * k2b[h2p]
    x1, x2 = _cross(e1, e2, Wc2)
    s2 = jnp.concatenate([_readout(x1, Wr2), _readout(x2, Wr2)], axis=1)

    g1o, g2o = he_layer(x1, x2, h1p2, a1p2, h2p2, a2p2, K2, KP2, W3)
    e1, _, _ = _pool(g1o, K3, p3, K3)
    e2, _, _ = _pool(g2o, K3, p3, K3)
    x1, x2 = _cross(e1, e2, Wc3)
    s3 = jnp.concatenate([_readout(x1, Wr3), _readout(x2, Wr3)], axis=1)

    scores = jnp.concatenate([s0, s1, s2, s3], axis=1)
    hmid = _leaky(scores @ Wm1 + bm1)
    return hmid @ Wm2 + bm2


# speedup vs baseline: 1.0858x; 1.0050x over previous
import functools

import jax
import jax.numpy as jnp
from jax import lax
from jax.experimental import pallas as pl
from jax.experimental.pallas import tpu as pltpu
from jax.experimental.pallas import tpu_sc as plsc

EPS = 1e-9
NC, NS, L = 2, 16, 16
CHUNK = 512
NHID = 64


def _dyn_splat(v, i):
    idx = jnp.full((L,), i, jnp.int32)
    return lax.gather(
        v, idx[:, None],
        lax.GatherDimensionNumbers(offset_dims=(), collapsed_slice_dims=(0,),
                                   start_index_map=(0,)),
        (1,), mode=lax.GatherScatterMode.PROMISE_IN_BOUNDS)


def _chunk_ranges(total, maxc):
    offs, off = [], 0
    while off < total:
        size = maxc
        while size > total - off:
            size //= 2
        offs.append((off, size))
        off += size
    return offs


def _qrsqrt(x):
    i = lax.bitcast_convert_type(x, jnp.int32)
    i = jnp.int32(0x5F3759DF) - lax.shift_right_logical(i, 1)
    y = lax.bitcast_convert_type(i, jnp.float32)
    for _ in range(3):
        y = y * (1.5 - 0.5 * x * y * y)
    return y


@functools.lru_cache(maxsize=None)
def _make_fused_kernel(mode, SP, MID, TP, TOT):
    nch = TOT // (NS * CHUNK)
    spr = SP // NS
    SD = TP if mode == "gcn" else MID
    secs = [SP // NS, SD // NS, TP // NS] + ([MID // NS] if MID else [])
    BN = max(secs)
    mesh = plsc.VectorSubcoreMesh(core_axis_name="c", subcore_axis_name="s")

    scratch = [
        pltpu.VMEM((2, CHUNK), jnp.int32),
        pltpu.VMEM((2, CHUNK), jnp.int32),
        pltpu.VMEM((2, CHUNK), jnp.float32),
        pltpu.VMEM((CHUNK,), jnp.float32),
        pltpu.VMEM((CHUNK,), jnp.float32),
        pltpu.VMEM((CHUNK, NHID), jnp.float32),
        pltpu.VMEM((BN,), jnp.float32),
        pltpu.VMEM_SHARED((SP, NHID), jnp.float32),
        pltpu.VMEM_SHARED((TP if mode == "gcn" else MID, NHID),
                          jnp.float32),
        pltpu.VMEM_SHARED((SP,), jnp.float32),
        pltpu.VMEM_SHARED((SD,), jnp.float32),
        pltpu.SemaphoreType.DMA,
        pltpu.SemaphoreType.DMA,
        pltpu.SemaphoreType.DMA,
        pltpu.SemaphoreType.DMA,
    ]

    @functools.partial(
        pl.kernel,
        out_type=jax.ShapeDtypeStruct((NC * TP, NHID), jnp.float32),
        mesh=mesh,
        compiler_params=pltpu.CompilerParams(
            needs_layout_passes=False, use_tc_tiling_on_sc=False),
        scratch_types=scratch,
    )
    def kern(x_hbm, g_hbm, s_hbm, w_hbm, out_hbm, *refs):
        (g_buf, s_buf, w_buf, gs_v, ss_v, rows, bounce,
         xA, B, degG, degS, sem, semg, sems, semw) = refs
        cid = lax.axis_index("c")
        sid = lax.axis_index("s")

        def zero_vec(buf, total):
            def body(k, c):
                buf[pl.ds(k * L, L)] = jnp.zeros((L,), jnp.float32)
                return c
            lax.fori_loop(0, total // L, body, 0)

        def zero_rows():
            def body(k, c):
                for j in range(NHID // L):
                    rows[k, pl.ds(j * L, L)] = jnp.zeros((L,), jnp.float32)
                return c
            lax.fori_loop(0, CHUNK, body, 0)

        def zero_table(tab, rows_n):
            for off, size in _chunk_ranges(rows_n, CHUNK):
                pltpu.sync_copy(rows.at[pl.ds(0, size)],
                                tab.at[pl.ds(sid * rows_n + off, size)])

        def zero_scalar(tab, n_sec):
            pltpu.sync_copy(bounce.at[pl.ds(0, n_sec)],
                            tab.at[pl.ds(sid * n_sec, n_sec)])

        def transform(tab, n_sec, fn):
            pltpu.sync_copy(tab.at[pl.ds(sid * n_sec, n_sec)],
                            bounce.at[pl.ds(0, n_sec)])

            def body(k, c):
                v = bounce[pl.ds(k * L, L)]
                bounce[pl.ds(k * L, L)] = fn(v + EPS)
                return c
            lax.fori_loop(0, n_sec // L, body, 0)
            pltpu.sync_copy(bounce.at[pl.ds(0, n_sec)],
                            tab.at[pl.ds(sid * n_sec, n_sec)])

        def _stream_dmas(it, slot):
            base = cid * TOT + (it * NS + sid) * CHUNK
            return (
                pltpu.make_async_copy(g_hbm.at[pl.ds(base, CHUNK)],
                                      g_buf.at[slot], semg),
                pltpu.make_async_copy(s_hbm.at[pl.ds(base, CHUNK)],
                                      s_buf.at[slot], sems),
                pltpu.make_async_copy(w_hbm.at[pl.ds(base, CHUNK)],
                                      w_buf.at[slot], semw))

        def stream_loop(proc):
            def issue(it, slot):
                for d in _stream_dmas(it, slot):
                    d.start()

            def waitall(it, slot):
                for d in _stream_dmas(it, slot):
                    d.wait()

            nhalf = nch // 2
            issue(0, 0)

            def body(i2, c):
                a = 2 * i2
                issue(a + 1, 1)
                waitall(a, 0)
                proc(0)

                @pl.when(i2 + 1 < nhalf)
                def _():
                    issue(a + 2, 0)

                waitall(a + 1, 1)
                proc(1)
                return c
            lax.fori_loop(0, nhalf, body, 0)

        def feature_pass(src_tab, dst_tab, swap, use_gs, ssT):
            def proc(slot):
                gb, sb, wb = g_buf.at[slot], s_buf.at[slot], w_buf.at[slot]
                gi = sb if swap else gb
                si = gb if swap else sb
                cp = pltpu.async_copy(src_tab.at[gi], rows, sem)
                if use_gs:
                    pltpu.sync_copy(degG.at[gi], gs_v)
                pltpu.sync_copy(ssT.at[si], ss_v)
                cp.wait()

                def scale(k, c2):
                    b16 = k * L
                    wv = w_buf[slot, pl.ds(b16, L)] * ss_v[pl.ds(b16, L)]
                    if use_gs:
                        wv = wv * gs_v[pl.ds(b16, L)]
                    for i in range(L):
                        spl = _dyn_splat(wv, i)
                        for j in range(NHID // L):
                            sl = pl.ds(j * L, L)
                            rows[b16 + i, sl] = rows[b16 + i, sl] * spl
                    return c2

                lax.fori_loop(0, CHUNK // L, scale, 0)
                pltpu.sync_copy(rows, dst_tab.at[si], add=True)
            stream_loop(proc)

        def leaky_sweep(tab, rows_n):
            for off, size in _chunk_ranges(rows_n, CHUNK):
                pltpu.sync_copy(tab.at[pl.ds(sid * rows_n + off, size)],
                                rows.at[pl.ds(0, size)])

                def body(r, c):
                    for j in range(NHID // L):
                        sl = pl.ds(j * L, L)
                        v = rows[r, sl]
                        rows[r, sl] = jnp.where(v > 0, v, 0.2 * v)
                    return c
                lax.fori_loop(0, size, body, 0)
                pltpu.sync_copy(rows.at[pl.ds(0, size)],
                                tab.at[pl.ds(sid * rows_n + off, size)])

        zero_vec(bounce, BN)
        zero_scalar(degG, SP // NS)
        zero_scalar(degS, SD // NS)
        for off, size in _chunk_ranges(spr, CHUNK):
            pltpu.sync_copy(x_hbm.at[pl.ds(cid * SP + sid * spr + off, size)],
                            rows.at[pl.ds(0, size)])
            pltpu.sync_copy(rows.at[pl.ds(0, size)],
                            xA.at[pl.ds(sid * spr + off, size)])
        zero_rows()
        zero_table(B, (TP if mode == "gcn" else MID) // NS)
        plsc.subcore_barrier()

        def deg_proc(slot):
            pltpu.sync_copy(w_buf.at[slot], degG.at[g_buf.at[slot]],
                            add=True)
            pltpu.sync_copy(w_buf.at[slot], degS.at[s_buf.at[slot]],
                            add=True)
        stream_loop(deg_proc)
        plsc.subcore_barrier()

        norm = _qrsqrt if mode == "gcn" else (lambda v: 1.0 / v)
        transform(degG, SP // NS, norm)
        transform(degS, SD // NS, norm)
        plsc.subcore_barrier()

        if mode == "gcn":
            feature_pass(xA, B, swap=False, use_gs=True, ssT=degS)
            out_tab = B
        elif mode == "he":
            feature_pass(xA, B, swap=False, use_gs=False, ssT=degS)
            plsc.subcore_barrier()
            zero_rows()
            zero_table(xA, SP // NS)
            plsc.subcore_barrier()
            feature_pass(B, xA, swap=True, use_gs=False, ssT=degG)
            out_tab = xA
        else:
            feature_pass(xA, B, swap=False, use_gs=False, ssT=degS)
            plsc.subcore_barrier()
            zero_rows()
            zero_table(xA, SP // NS)
            plsc.subcore_barrier()
            feature_pass(B, xA, swap=True, use_gs=False, ssT=degG)
            plsc.subcore_barrier()
            leaky_sweep(xA, SP // NS)
            zero_rows()
            zero_table(B, MID // NS)
            plsc.subcore_barrier()
            feature_pass(xA, B, swap=False, use_gs=False, ssT=degS)
            out_tab = B

        plsc.subcore_barrier()
        rps = TP // NS
        for off, size in _chunk_ranges(rps, CHUNK):
            pltpu.sync_copy(out_tab.at[pl.ds(sid * rps + off, size)],
                            rows.at[pl.ds(0, size)])
            pltpu.sync_copy(
                rows.at[pl.ds(0, size)],
                out_hbm.at[pl.ds(cid * TP + sid * rps + off, size)])

    return kern


def _pad1(g, s, bw, SP, SD, TOT):
    npad = TOT - g.shape[0]
    if npad:
        ar = jnp.arange(npad, dtype=jnp.int32)
        g = jnp.concatenate([g, ar % SP])
        s = jnp.concatenate([s, ar % SD])
        bw = jnp.concatenate([bw, jnp.zeros((npad,), bw.dtype)])
    return g, s, bw


def _xtab(x1, x2, P):
    o1 = jnp.zeros((P, x1.shape[1]), x1.dtype).at[:x1.shape[0]].set(x1)
    o2 = jnp.zeros((P, x2.shape[1]), x2.dtype).at[:x2.shape[0]].set(x2)
    return jnp.concatenate([o1, o2], axis=0)


def _fused(mode, x1, x2, pairs1, pairs2, SP, MID, TP, TOT):
    SD = TP if mode == "gcn" else MID
    g1, s1, w1 = _pad1(*pairs1, SP, SD, TOT)
    g2, s2, w2 = _pad1(*pairs2, SP, SD, TOT)
    g = jnp.concatenate([g1, g2])
    s = jnp.concatenate([s1, s2])
    w = jnp.concatenate([w1, w2])
    x = _xtab(x1, x2, SP)
    kern = _make_fused_kernel(mode, SP, MID, TP, TOT)
    return kern(x, g, s, w)


def _leaky(x):
    return jnp.where(x > 0, x, 0.2 * x)


def _readout(x, Wr):
    m = jnp.mean(x, axis=0, keepdims=True)
    gate = jax.nn.sigmoid(x @ Wr @ m.T)
    return jnp.sum(gate * x, axis=0, keepdims=True)


def _cross(x1, x2, W):
    a12 = jax.nn.softmax((x1 @ W) @ x2.T, axis=1)
    a21 = jax.nn.softmax((x2 @ W) @ x1.T, axis=1)
    return a12 @ x2, a21 @ x1


def _pool(ef, k, p, kp):
    score = jnp.tanh(ef @ p / (jnp.linalg.norm(p) + EPS))
    vals, idx = lax.top_k(score, k)
    pooled = ef[idx] * vals[:, None]
    num = ef.shape[0]
    mapping = (jnp.arange(num, dtype=jnp.int32) % kp).at[idx].set(
        jnp.arange(k, dtype=jnp.int32))
    keep = jnp.zeros((num,), ef.dtype).at[idx].set(1.0)
    return pooled, mapping, keep


def kernel(features_1, edge_index_1, edge_attr_1, batch_1, features_2,
           edge_index_2, edge_attr_2, batch_2, W0, b0, W1, W2, W3, Wc1, Wc2,
           Wc3, p1, p2, p3, Wr0, Wr1, Wr2, Wr3, Wm1, bm1, Wm2, bm2):
    n = features_1.shape[0]
    K1 = int(0.2 * n); K2 = K1 // 2; K3 = K2 // 2
    NP = -(-n // 1024) * 1024
    KP1 = -(-K1 // 1024) * 1024
    KP2 = -(-K2 // 1024) * 1024
    E_ = edge_index_1.shape[1]
    M_ = E_ + n
    GTOT = -(-E_ // (2 * NS * CHUNK)) * (2 * NS * CHUNK)
    FTOT = -(-M_ // (2 * NS * CHUNK)) * (2 * NS * CHUNK)
    src1, dst1 = edge_index_1[0], edge_index_1[1]
    src2, dst2 = edge_index_2[0], edge_index_2[1]
    ew1, ew2 = edge_attr_1, edge_attr_2

    h1 = features_1 @ W0
    h2 = features_2 @ W0
    out = _fused("gcn", h1, h2, (src1, dst1, ew1), (src2, dst2, ew2),
                 NP, None, NP, GTOT)
    f1 = _leaky(out[:n] + b0)
    f2 = _leaky(out[NP:NP + n] + b0)
    s0 = jnp.concatenate([_readout(f1, Wr0), _readout(f2, Wr0)], axis=1)

    ar_n = jnp.arange(n, dtype=jnp.int32)
    n1 = jnp.concatenate([src1, ar_n]); h1i = jnp.concatenate([dst1, ar_n])
    a1 = jnp.concatenate([ew1, jnp.ones((n,), jnp.float32)])
    n2 = jnp.concatenate([src2, ar_n]); h2i = jnp.concatenate([dst2, ar_n])
    a2 = jnp.concatenate([ew2, jnp.ones((n,), jnp.float32)])

    hh1 = f1 @ W1; hh2 = f2 @ W1
    ef = _fused("hg", hh1, hh2, (n1, h1i, a1), (n2, h2i, a2),
                NP, NP, NP, FTOT)
    ef1 = ef[:n]; ef2 = ef[NP:NP + n]

    e1, map1, keep1 = _pool(ef1, K1, p1, KP1)
    e2, map2, keep2 = _pool(ef2, K1, p1, KP1)
    h1p = map1[h1i]; a1p = a1 * keep1[h1i]
    h2p = map2[h2i]; a2p = a2 * keep2[h2i]
    x1, x2 = _cross(e1, e2, Wc1)
    s1 = jnp.concatenate([_readout(x1, Wr1), _readout(x2, Wr1)], axis=1)

    def he_layer(x1, x2, h1p, a1p, h2p, a2p, K, KP, W):
        out = _fused("he", x1, x2, (h1p, n1, a1p), (h2p, n2, a2p),
                     KP, NP, KP, FTOT)
        o1 = _leaky(out[:K] @ W)
        o2 = _leaky(out[KP:KP + K] @ W)
        return o1, o2

    g1o, g2o = he_layer(x1, x2, h1p, a1p, h2p, a2p, K1, KP1, W2)
    e1, m1b, k1b = _pool(g1o, K2, p2, KP2)
    e2, m2b, k2b = _pool(g2o, K2, p2, KP2)

    def _ext(m, keep, total, kp):
        num = m.shape[0]
        extra = jnp.arange(num, total, dtype=jnp.int32) % kp
        return (jnp.concatenate([m, extra]),
                jnp.concatenate([keep, jnp.zeros((total - num,), keep.dtype)]))

    m1b, k1b = _ext(m1b, k1b, KP1, KP2)
    m2b, k2b = _ext(m2b, k2b, KP1, KP2)
    h1p2 = m1b[h1p]; a1p2 = a1p * k1b[h1p]
    h2p2 = m2b[h2p]; a2p2 = a2p * k2b[h2p]
    x1, x2 = _cross(e1, e2, Wc2)
    s2 = jnp.concatenate([_readout(x1, Wr2), _readout(x2, Wr2)], axis=1)

    g1o, g2o = he_layer(x1, x2, h1p2, a1p2, h2p2, a2p2, K2, KP2, W3)
    e1, _, _ = _pool(g1o, K3, p3, K3)
    e2, _, _ = _pool(g2o, K3, p3, K3)
    x1, x2 = _cross(e1, e2, Wc3)
    s3 = jnp.concatenate([_readout(x1, Wr3), _readout(x2, Wr3)], axis=1)

    scores = jnp.concatenate([s0, s1, s2, s3], axis=1)
    hmid = _leaky(scores @ Wm1 + bm1)
    return hmid @ Wm2 + bm2


# speedup vs baseline: 5.9033x; 5.4367x over previous
import functools

import jax
import jax.numpy as jnp
from jax import lax
from jax.experimental import pallas as pl
from jax.experimental.pallas import tpu as pltpu
from jax.experimental.pallas import tpu_sc as plsc

EPS = 1e-9
NC, NS, L = 2, 16, 16
CHUNK = 512
NHID = 64


def _dyn_splat(v, i):
    idx = jnp.full((L,), i, jnp.int32)
    return lax.gather(
        v, idx[:, None],
        lax.GatherDimensionNumbers(offset_dims=(), collapsed_slice_dims=(0,),
                                   start_index_map=(0,)),
        (1,), mode=lax.GatherScatterMode.PROMISE_IN_BOUNDS)


def _chunk_ranges(total, maxc):
    offs, off = [], 0
    while off < total:
        size = maxc
        while size > total - off:
            size //= 2
        offs.append((off, size))
        off += size
    return offs


def _qrsqrt(x):
    i = lax.bitcast_convert_type(x, jnp.int32)
    i = jnp.int32(0x5F3759DF) - lax.shift_right_logical(i, 1)
    y = lax.bitcast_convert_type(i, jnp.float32)
    for _ in range(3):
        y = y * (1.5 - 0.5 * x * y * y)
    return y


@functools.lru_cache(maxsize=None)
def _make_fused_kernel(mode, SP, MID, TP, TOT):
    nch = TOT // (NS * CHUNK)
    spr = SP // NS
    SD = TP if mode == "gcn" else MID
    secs = [SP // NS, SD // NS, TP // NS] + ([MID // NS] if MID else [])
    BN = max(secs)
    mesh = plsc.VectorSubcoreMesh(core_axis_name="c", subcore_axis_name="s")

    scratch = [
        pltpu.VMEM((2, CHUNK), jnp.int32),
        pltpu.VMEM((2, CHUNK), jnp.int32),
        pltpu.VMEM((2, CHUNK), jnp.float32),
        pltpu.VMEM((CHUNK,), jnp.float32),
        pltpu.VMEM((CHUNK,), jnp.float32),
        pltpu.VMEM((CHUNK, NHID), jnp.float32),
        pltpu.VMEM((BN,), jnp.float32),
        pltpu.VMEM_SHARED((SP, NHID), jnp.float32),
        pltpu.VMEM_SHARED((TP if mode == "gcn" else MID, NHID),
                          jnp.float32),
        pltpu.VMEM_SHARED((SP,), jnp.float32),
        pltpu.VMEM_SHARED((SD,), jnp.float32),
    ]
    if mode == "he":
        scratch += [
            pltpu.VMEM((BN,), jnp.int32),
            pltpu.VMEM((CHUNK,), jnp.int32),
            pltpu.VMEM_SHARED((MID,), jnp.int32),
            pltpu.VMEM_SHARED((MID,), jnp.float32),
        ]
    scratch += [
        pltpu.SemaphoreType.DMA,
        pltpu.SemaphoreType.DMA,
        pltpu.SemaphoreType.DMA,
        pltpu.SemaphoreType.DMA,
    ]

    def _body(x_hbm, g_hbm, s_hbm, w_hbm, mp_hbm, kp_hbm, out_hbm, refs):
        if mode == "he":
            (g_buf, s_buf, w_buf, gs_v, ss_v, rows, bounce,
             xA, B, degG, degS, ib, hidx, mp_sh, kp_sh,
             sem, semg, sems, semw) = refs
        else:
            (g_buf, s_buf, w_buf, gs_v, ss_v, rows, bounce,
             xA, B, degG, degS, sem, semg, sems, semw) = refs
        cid = lax.axis_index("c")
        sid = lax.axis_index("s")

        def zero_vec(buf, total):
            def body(k, c):
                buf[pl.ds(k * L, L)] = jnp.zeros((L,), jnp.float32)
                return c
            lax.fori_loop(0, total // L, body, 0)

        def zero_rows():
            def body(k, c):
                for j in range(NHID // L):
                    rows[k, pl.ds(j * L, L)] = jnp.zeros((L,), jnp.float32)
                return c
            lax.fori_loop(0, CHUNK, body, 0)

        def zero_table(tab, rows_n):
            for off, size in _chunk_ranges(rows_n, CHUNK):
                pltpu.sync_copy(rows.at[pl.ds(0, size)],
                                tab.at[pl.ds(sid * rows_n + off, size)])

        def zero_scalar(tab, n_sec):
            pltpu.sync_copy(bounce.at[pl.ds(0, n_sec)],
                            tab.at[pl.ds(sid * n_sec, n_sec)])

        def transform(tab, n_sec, fn):
            pltpu.sync_copy(tab.at[pl.ds(sid * n_sec, n_sec)],
                            bounce.at[pl.ds(0, n_sec)])

            def body(k, c):
                v = bounce[pl.ds(k * L, L)]
                bounce[pl.ds(k * L, L)] = fn(v + EPS)
                return c
            lax.fori_loop(0, n_sec // L, body, 0)
            pltpu.sync_copy(bounce.at[pl.ds(0, n_sec)],
                            tab.at[pl.ds(sid * n_sec, n_sec)])

        def _stream_dmas(it, slot):
            base = cid * TOT + (it * NS + sid) * CHUNK
            return (
                pltpu.make_async_copy(g_hbm.at[pl.ds(base, CHUNK)],
                                      g_buf.at[slot], semg),
                pltpu.make_async_copy(s_hbm.at[pl.ds(base, CHUNK)],
                                      s_buf.at[slot], sems),
                pltpu.make_async_copy(w_hbm.at[pl.ds(base, CHUNK)],
                                      w_buf.at[slot], semw))

        def stream_loop(proc):
            def issue(it, slot):
                for d in _stream_dmas(it, slot):
                    d.start()

            def waitall(it, slot):
                for d in _stream_dmas(it, slot):
                    d.wait()

            nhalf = nch // 2
            issue(0, 0)

            def body(i2, c):
                a = 2 * i2
                issue(a + 1, 1)
                waitall(a, 0)
                proc(0)

                @pl.when(i2 + 1 < nhalf)
                def _():
                    issue(a + 2, 0)

                waitall(a + 1, 1)
                proc(1)
                return c
            lax.fori_loop(0, nhalf, body, 0)

        def feature_pass(src_tab, dst_tab, swap, use_gs, ssT, use_map=False):
            def proc(slot):
                gb, sb = g_buf.at[slot], s_buf.at[slot]
                if use_map:
                    pltpu.sync_copy(mp_sh.at[sb], hidx)
                    pltpu.sync_copy(kp_sh.at[sb], gs_v)
                    hs = hidx
                else:
                    hs = sb
                gi = hs if swap else gb
                si = gb if swap else hs
                cp = pltpu.async_copy(src_tab.at[gi], rows, sem)
                if use_gs:
                    pltpu.sync_copy(degG.at[gi], gs_v)
                pltpu.sync_copy(ssT.at[si], ss_v)
                cp.wait()

                def scale(k, c2):
                    b16 = k * L
                    wv = w_buf[slot, pl.ds(b16, L)] * ss_v[pl.ds(b16, L)]
                    if use_gs or use_map:
                        wv = wv * gs_v[pl.ds(b16, L)]
                    for i in range(L):
                        spl = _dyn_splat(wv, i)
                        for j in range(NHID // L):
                            sl = pl.ds(j * L, L)
                            rows[b16 + i, sl] = rows[b16 + i, sl] * spl
                    return c2

                lax.fori_loop(0, CHUNK // L, scale, 0)
                pltpu.sync_copy(rows, dst_tab.at[si], add=True)
            stream_loop(proc)

        def leaky_sweep(tab, rows_n):
            for off, size in _chunk_ranges(rows_n, CHUNK):
                pltpu.sync_copy(tab.at[pl.ds(sid * rows_n + off, size)],
                                rows.at[pl.ds(0, size)])

                def body(r, c):
                    for j in range(NHID // L):
                        sl = pl.ds(j * L, L)
                        v = rows[r, sl]
                        rows[r, sl] = jnp.where(v > 0, v, 0.2 * v)
                    return c
                lax.fori_loop(0, size, body, 0)
                pltpu.sync_copy(rows.at[pl.ds(0, size)],
                                tab.at[pl.ds(sid * rows_n + off, size)])

        if mode == "he":
            msec = MID // NS
            pltpu.sync_copy(mp_hbm.at[pl.ds(cid * MID + sid * msec, msec)],
                            ib.at[pl.ds(0, msec)])
            pltpu.sync_copy(ib.at[pl.ds(0, msec)],
                            mp_sh.at[pl.ds(sid * msec, msec)])
            pltpu.sync_copy(kp_hbm.at[pl.ds(cid * MID + sid * msec, msec)],
                            bounce.at[pl.ds(0, msec)])
            pltpu.sync_copy(bounce.at[pl.ds(0, msec)],
                            kp_sh.at[pl.ds(sid * msec, msec)])
        zero_vec(bounce, BN)
        zero_scalar(degG, SP // NS)
        zero_scalar(degS, SD // NS)
        for off, size in _chunk_ranges(spr, CHUNK):
            pltpu.sync_copy(x_hbm.at[pl.ds(cid * SP + sid * spr + off, size)],
                            rows.at[pl.ds(0, size)])
            pltpu.sync_copy(rows.at[pl.ds(0, size)],
                            xA.at[pl.ds(sid * spr + off, size)])
        zero_rows()
        zero_table(B, (TP if mode == "gcn" else MID) // NS)
        plsc.subcore_barrier()

        if mode == "he":
            def deg_proc(slot):
                sb = s_buf.at[slot]
                pltpu.sync_copy(mp_sh.at[sb], hidx)
                pltpu.sync_copy(kp_sh.at[sb], gs_v)

                def wmul(k, c):
                    d = pl.ds(k * L, L)
                    ss_v[d] = w_buf[slot, d] * gs_v[d]
                    return c
                lax.fori_loop(0, CHUNK // L, wmul, 0)
                pltpu.sync_copy(ss_v, degG.at[hidx], add=True)
                pltpu.sync_copy(ss_v, degS.at[g_buf.at[slot]], add=True)
        else:
            def deg_proc(slot):
                pltpu.sync_copy(w_buf.at[slot], degG.at[g_buf.at[slot]],
                                add=True)
                pltpu.sync_copy(w_buf.at[slot], degS.at[s_buf.at[slot]],
                                add=True)
        stream_loop(deg_proc)
        plsc.subcore_barrier()

        norm = _qrsqrt if mode == "gcn" else (lambda v: 1.0 / v)
        transform(degG, SP // NS, norm)
        transform(degS, SD // NS, norm)
        plsc.subcore_barrier()

        if mode == "gcn":
            feature_pass(xA, B, swap=False, use_gs=True, ssT=degS)
            out_tab = B
        elif mode == "he":
            feature_pass(xA, B, swap=True, use_gs=False, ssT=degS,
                         use_map=True)
            plsc.subcore_barrier()
            zero_rows()
            zero_table(xA, SP // NS)
            plsc.subcore_barrier()
            feature_pass(B, xA, swap=False, use_gs=False, ssT=degG,
                         use_map=True)
            out_tab = xA
        else:
            feature_pass(xA, B, swap=False, use_gs=False, ssT=degS)
            plsc.subcore_barrier()
            zero_rows()
            zero_table(xA, SP // NS)
            plsc.subcore_barrier()
            feature_pass(B, xA, swap=True, use_gs=False, ssT=degG)
            plsc.subcore_barrier()
            leaky_sweep(xA, SP // NS)
            zero_rows()
            zero_table(B, MID // NS)
            plsc.subcore_barrier()
            feature_pass(xA, B, swap=False, use_gs=False, ssT=degS)
            out_tab = B

        plsc.subcore_barrier()
        rps = TP // NS
        for off, size in _chunk_ranges(rps, CHUNK):
            pltpu.sync_copy(out_tab.at[pl.ds(sid * rps + off, size)],
                            rows.at[pl.ds(0, size)])
            pltpu.sync_copy(
                rows.at[pl.ds(0, size)],
                out_hbm.at[pl.ds(cid * TP + sid * rps + off, size)])

    pk = functools.partial(
        pl.kernel,
        out_type=jax.ShapeDtypeStruct((NC * TP, NHID), jnp.float32),
        mesh=mesh,
        compiler_params=pltpu.CompilerParams(
            needs_layout_passes=False, use_tc_tiling_on_sc=False),
        scratch_types=scratch)
    if mode == "he":
        @pk
        def kern(x_hbm, g_hbm, s_hbm, w_hbm, mp_hbm, kp_hbm, out_hbm, *refs):
            _body(x_hbm, g_hbm, s_hbm, w_hbm, mp_hbm, kp_hbm, out_hbm, refs)
    else:
        @pk
        def kern(x_hbm, g_hbm, s_hbm, w_hbm, out_hbm, *refs):
            _body(x_hbm, g_hbm, s_hbm, w_hbm, None, None, out_hbm, refs)

    return kern


def _pad1(g, s, bw, SP, SD, TOT):
    npad = TOT - g.shape[0]
    if npad:
        ar = jnp.arange(npad, dtype=jnp.int32)
        g = jnp.concatenate([g, ar % SP])
        s = jnp.concatenate([s, ar % SD])
        bw = jnp.concatenate([bw, jnp.zeros((npad,), bw.dtype)])
    return g, s, bw


def _xtab(x1, x2, P):
    o1 = jnp.zeros((P, x1.shape[1]), x1.dtype).at[:x1.shape[0]].set(x1)
    o2 = jnp.zeros((P, x2.shape[1]), x2.dtype).at[:x2.shape[0]].set(x2)
    return jnp.concatenate([o1, o2], axis=0)


def _fused(mode, x1, x2, g, s, w, SP, MID, TP, TOT, mp=None, kp=None):
    x = _xtab(x1, x2, SP)
    kern = _make_fused_kernel(mode, SP, MID, TP, TOT)
    if mode == "he":
        return kern(x, g, s, w, mp, kp)
    return kern(x, g, s, w)


def _leaky(x):
    return jnp.where(x > 0, x, 0.2 * x)


def _readout(x, Wr):
    m = jnp.mean(x, axis=0, keepdims=True)
    gate = jax.nn.sigmoid(x @ Wr @ m.T)
    return jnp.sum(gate * x, axis=0, keepdims=True)


def _cross(x1, x2, W):
    a12 = jax.nn.softmax((x1 @ W) @ x2.T, axis=1)
    a21 = jax.nn.softmax((x2 @ W) @ x1.T, axis=1)
    return a12 @ x2, a21 @ x1


def _pool(ef, k, p, kp):
    score = jnp.tanh(ef @ p / (jnp.linalg.norm(p) + EPS))
    vals, idx = lax.top_k(score, k)
    pooled = ef[idx] * vals[:, None]
    num = ef.shape[0]
    mapping = (jnp.arange(num, dtype=jnp.int32) % kp).at[idx].set(
        jnp.arange(k, dtype=jnp.int32))
    keep = jnp.zeros((num,), ef.dtype).at[idx].set(1.0)
    return pooled, mapping, keep


def kernel(features_1, edge_index_1, edge_attr_1, batch_1, features_2,
           edge_index_2, edge_attr_2, batch_2, W0, b0, W1, W2, W3, Wc1, Wc2,
           Wc3, p1, p2, p3, Wr0, Wr1, Wr2, Wr3, Wm1, bm1, Wm2, bm2):
    n = features_1.shape[0]
    K1 = int(0.2 * n); K2 = K1 // 2; K3 = K2 // 2
    NP = -(-n // 1024) * 1024
    KP1 = -(-K1 // 1024) * 1024
    KP2 = -(-K2 // 1024) * 1024
    E_ = edge_index_1.shape[1]
    M_ = E_ + n
    GTOT = -(-E_ // (2 * NS * CHUNK)) * (2 * NS * CHUNK)
    FTOT = -(-M_ // (2 * NS * CHUNK)) * (2 * NS * CHUNK)
    src1, dst1 = edge_index_1[0], edge_index_1[1]
    src2, dst2 = edge_index_2[0], edge_index_2[1]
    ew1, ew2 = edge_attr_1, edge_attr_2

    def _streams(pairs1, pairs2, SP, SD, TOT):
        g1, s1, w1 = _pad1(*pairs1, SP, SD, TOT)
        g2, s2, w2 = _pad1(*pairs2, SP, SD, TOT)
        return (jnp.concatenate([g1, g2]), jnp.concatenate([s1, s2]),
                jnp.concatenate([w1, w2]))

    h1 = features_1 @ W0
    h2 = features_2 @ W0
    gG, sG, wG = _streams((src1, dst1, ew1), (src2, dst2, ew2), NP, NP, GTOT)
    out = _fused("gcn", h1, h2, gG, sG, wG, NP, None, NP, GTOT)
    f1 = _leaky(out[:n] + b0)
    f2 = _leaky(out[NP:NP + n] + b0)
    s0 = jnp.concatenate([_readout(f1, Wr0), _readout(f2, Wr0)], axis=1)

    ar_n = jnp.arange(n, dtype=jnp.int32)
    n1 = jnp.concatenate([src1, ar_n]); h1i = jnp.concatenate([dst1, ar_n])
    a1 = jnp.concatenate([ew1, jnp.ones((n,), jnp.float32)])
    n2 = jnp.concatenate([src2, ar_n]); h2i = jnp.concatenate([dst2, ar_n])
    a2 = jnp.concatenate([ew2, jnp.ones((n,), jnp.float32)])
    gI, sI, wI = _streams((n1, h1i, a1), (n2, h2i, a2), NP, NP, FTOT)

    hh1 = f1 @ W1; hh2 = f2 @ W1
    ef = _fused("hg", hh1, hh2, gI, sI, wI, NP, NP, NP, FTOT)
    ef1 = ef[:n]; ef2 = ef[NP:NP + n]

    e1, map1, keep1 = _pool(ef1, K1, p1, KP1)
    e2, map2, keep2 = _pool(ef2, K1, p1, KP1)
    x1, x2 = _cross(e1, e2, Wc1)
    s1 = jnp.concatenate([_readout(x1, Wr1), _readout(x2, Wr1)], axis=1)

    def _mpkp(m1, k1, m2, k2, kp):
        extra = jnp.arange(n, NP, dtype=jnp.int32) % kp
        zpad = jnp.zeros((NP - n,), jnp.float32)
        return (jnp.concatenate([m1, extra, m2, extra]),
                jnp.concatenate([k1, zpad, k2, zpad]))

    def he_layer(x1, x2, mp, kp, K, KP, W):
        out = _fused("he", x1, x2, gI, sI, wI, KP, NP, KP, FTOT,
                     mp=mp, kp=kp)
        o1 = _leaky(out[:K] @ W)
        o2 = _leaky(out[KP:KP + K] @ W)
        return o1, o2

    mp, kp = _mpkp(map1, keep1, map2, keep2, KP1)
    g1o, g2o = he_layer(x1, x2, mp, kp, K1, KP1, W2)
    e1, m1b, k1b = _pool(g1o, K2, p2, KP2)
    e2, m2b, k2b = _pool(g2o, K2, p2, KP2)

    def _ext(m, keep, total, kp_):
        num = m.shape[0]
        extra = jnp.arange(num, total, dtype=jnp.int32) % kp_
        return (jnp.concatenate([m, extra]),
                jnp.concatenate([keep, jnp.zeros((total - num,), keep.dtype)]))

    m1b, k1b = _ext(m1b, k1b, KP1, KP2)
    m2b, k2b = _ext(m2b, k2b, KP1, KP2)
    mC1 = m1b[map1]; kC1 = keep1 * k1b[map1]
    mC2 = m2b[map2]; kC2 = keep2 * k2b[map2]
    x1, x2 = _cross(e1, e2, Wc2)
    s2 = jnp.concatenate([_readout(x1, Wr2), _readout(x2, Wr2)], axis=1)

    mp, kp = _mpkp(mC1, kC1, mC2, kC2, KP2)
    g1o, g2o = he_layer(x1, x2, mp, kp, K2, KP2, W3)
    e1, _, _ = _pool(g1o, K3, p3, K3)
    e2, _, _ = _pool(g2o, K3, p3, K3)
    x1, x2 = _cross(e1, e2, Wc3)
    s3 = jnp.concatenate([_readout(x1, Wr3), _readout(x2, Wr3)], axis=1)

    scores = jnp.concatenate([s0, s1, s2, s3], axis=1)
    hmid = _leaky(scores @ Wm1 + bm1)
    return hmid @ Wm2 + bm2
